# Initial kernel scaffold; baseline (speedup 1.0000x reference)
import jax
import jax.numpy as jnp
from jax.experimental import pallas as pl


def kernel(x, edge_index, dists, batch, Win, bin_, Wm1, bm1, Wm2, bm2, Wu1, bu1, Wu2, bu2, Wp, bp):
    raise NotImplementedError("write your pallas kernel here")


# speedup vs baseline: 1.5196x; 1.5196x over previous
import functools

import jax
import jax.numpy as jnp
from jax import lax
from jax.experimental import pallas as pl
from jax.experimental.pallas import tpu as pltpu
from jax.experimental.pallas import tpu_sc as plsc

N = 50000
E = 800000
D = 64
NC, NS = 2, 16
NW = NC * NS
CH = 128
RB = 2000
EB = 2000

_SC_MESH = plsc.VectorSubcoreMesh(
    core_axis_name="c", subcore_axis_name="s", num_cores=NC, num_subcores=NS)

HALF = N // 2
RSC = 25088
TILE_ROWS = RSC // NS
GARB = 25080
ZCH = 64


def _gather_body(ab_hbm, dst_hbm, src_hbm, out_hbm,
                 idxd, idxs, bufd, bufs, bufo, sem, sem2):
    c = lax.axis_index("c")
    s = lax.axis_index("s")
    w = s * NC + c
    n_chunks = E // CH
    jmax = -(-n_chunks // NW)

    def step(j, carry):
        ch = w + j * NW

        @pl.when(ch < n_chunks)
        def _():
            base = ch * CH
            pltpu.sync_copy(dst_hbm.at[pl.ds(base, CH)], idxd)
            pltpu.sync_copy(src_hbm.at[pl.ds(base, CH)], idxs)
            ca = pltpu.async_copy(ab_hbm.at[idxd], bufd, sem)
            cb = pltpu.async_copy(ab_hbm.at[idxs], bufs, sem2)
            ca.wait()
            cb.wait()

            def row(r, cc):
                for g in range(4):
                    sl = pl.ds(g * 16, 16)
                    sh = pl.ds(D + g * 16, 16)
                    bufo[r, sl] = bufd[r, sl] + bufs[r, sh]
                return cc

            lax.fori_loop(0, CH, row, 0)
            pltpu.sync_copy(bufo, out_hbm.at[pl.ds(base, CH)])

        return carry

    lax.fori_loop(0, jmax, step, 0)


_gather_add = pl.kernel(
    _gather_body,
    out_type=jax.ShapeDtypeStruct((E, D), jnp.float32),
    mesh=_SC_MESH,
    scratch_types=[
        pltpu.VMEM((CH,), jnp.int32),
        pltpu.VMEM((CH,), jnp.int32),
        pltpu.VMEM((CH, 2 * D), jnp.float32),
        pltpu.VMEM((CH, 2 * D), jnp.float32),
        pltpu.VMEM((CH, D), jnp.float32),
        pltpu.SemaphoreType.DMA,
        pltpu.SemaphoreType.DMA,
    ],
)


def _make_scatter(p):
    RQ = 12800
    GARB = 12704
    WB = 776

    def body(m_hbm, dst_hbm, out_hbm, idx, li, bufm, zbuf, acc):
        c = lax.axis_index("c")
        s = lax.axis_index("s")
        base_node = p * HALF + c * 12504
        own = 12504 - 8 * c
        n_chunks = E // CH
        jmax = -(-n_chunks // NS)

        def zrow(r, cc):
            for g in range(8):
                zbuf[r, pl.ds(g * 16, 16)] = jnp.zeros((16,), jnp.float32)
            return cc

        lax.fori_loop(0, ZCH, zrow, 0)

        def zs(k, cc):
            pltpu.sync_copy(zbuf, acc.at[pl.ds(s * (RQ // NS) + k * ZCH, ZCH)])
            return cc

        lax.fori_loop(0, RQ // NS // ZCH, zs, 0)
        ztail = RQ // NS - (RQ // NS // ZCH) * ZCH
        pltpu.sync_copy(zbuf.at[pl.ds(0, ztail)],
                        acc.at[pl.ds(s * (RQ // NS) + (RQ // NS // ZCH) * ZCH,
                                     ztail)])
        plsc.subcore_barrier()

        def step(j, cc):
            ch = s + j * NS

            @pl.when(ch < n_chunks)
            def _():
                base = ch * CH
                pltpu.sync_copy(dst_hbm.at[pl.ds(base, CH)], idx)
                pltpu.sync_copy(m_hbm.at[pl.ds(base, CH)], bufm)
                for g in range(8):
                    sl = pl.ds(g * 16, 16)
                    v = idx[sl] - base_node
                    oob = (v < 0) | (v >= own)
                    li[sl] = jnp.where(oob, GARB, v)
                pltpu.sync_copy(bufm, acc.at[li], add=True)

            return cc

        lax.fori_loop(0, jmax, step, 0)
        plsc.subcore_barrier()

        pltpu.sync_copy(acc.at[pl.ds(s * WB, WB)],
                        out_hbm.at[pl.ds(c * 12504 + s * WB, WB)])

        @pl.when((s == 0) & (c == 0))
        def _():
            pltpu.sync_copy(acc.at[pl.ds(NS * WB, 88)],
                            out_hbm.at[pl.ds(NS * WB, 88)])

        @pl.when((s == 0) & (c == 1))
        def _():
            pltpu.sync_copy(acc.at[pl.ds(NS * WB, 80)],
                            out_hbm.at[pl.ds(12504 + NS * WB, 80)])

    return pl.kernel(
        body,
        out_type=jax.ShapeDtypeStruct((HALF, 2 * D), jnp.float32),
        mesh=_SC_MESH,
        scratch_types=[
            pltpu.VMEM((CH,), jnp.int32),
            pltpu.VMEM((CH,), jnp.int32),
            pltpu.VMEM((CH, 2 * D), jnp.float32),
            pltpu.VMEM((ZCH, 2 * D), jnp.float32),
            pltpu.VMEM_SHARED((RQ, 2 * D), jnp.float32),
        ],
    )


_scatter_q0 = _make_scatter(0)
_scatter_q1 = _make_scatter(1)


def _node_in_body(x_ref, w_ref, par_ref, out_ref):
    out_ref[...] = jnp.dot(x_ref[...], w_ref[...],
                           preferred_element_type=jnp.float32) + par_ref[0:1, :]


def _node_in(xp, wp, par):
    return pl.pallas_call(
        _node_in_body,
        grid=(N // RB,),
        in_specs=[
            pl.BlockSpec((RB, 8), lambda i: (i, 0)),
            pl.BlockSpec((8, D), lambda i: (0, 0)),
            pl.BlockSpec((8, D), lambda i: (0, 0)),
        ],
        out_specs=pl.BlockSpec((RB, D), lambda i: (i, 0)),
        out_shape=jax.ShapeDtypeStruct((N, D), jnp.float32),
    )(xp, wp, par)


def _node_pre_body(e_ref, w_ref, ab_ref):
    ab_ref[...] = jnp.dot(e_ref[...], w_ref[...],
                          preferred_element_type=jnp.float32)


def _node_pre(emb, wcat):
    return pl.pallas_call(
        _node_pre_body,
        grid=(N // RB,),
        in_specs=[
            pl.BlockSpec((RB, D), lambda i: (i, 0)),
            pl.BlockSpec((D, 2 * D), lambda i: (0, 0)),
        ],
        out_specs=pl.BlockSpec((RB, 2 * D), lambda i: (i, 0)),
        out_shape=jax.ShapeDtypeStruct((N, 2 * D), jnp.float32),
    )(emb, wcat)


def _edge_mlp_body(tp_ref, d_ref, w2_ref, par_ref, out_ref):
    par = par_ref[...]
    t = jnp.maximum(tp_ref[...] + d_ref[...] * par[0:1, :] + par[1:2, :], 0.0)
    m = jnp.dot(t, w2_ref[...], preferred_element_type=jnp.float32)
    m = jnp.maximum(m + par[2:3, :], 0.0)
    out_ref[...] = jnp.concatenate([m, jnp.zeros_like(m)], axis=1)


def _edge_mlp(tp, dists, wm2, par):
    return pl.pallas_call(
        _edge_mlp_body,
        grid=(E // EB,),
        in_specs=[
            pl.BlockSpec((EB, D), lambda i: (i, 0)),
            pl.BlockSpec((EB, 1), lambda i: (i, 0)),
            pl.BlockSpec((D, D), lambda i: (0, 0)),
            pl.BlockSpec((8, D), lambda i: (0, 0)),
        ],
        out_specs=pl.BlockSpec((EB, 2 * D), lambda i: (i, 0)),
        out_shape=jax.ShapeDtypeStruct((E, 2 * D), jnp.float32),
    )(tp, dists, wm2, par)


def _node_upd_body(e_ref, g_ref, w1a_ref, w1b_ref, w2_ref, par_ref, out_ref):
    par = par_ref[...]
    e = e_ref[...]
    h = jnp.maximum(
        jnp.dot(e, w1a_ref[...], preferred_element_type=jnp.float32)
        + jnp.dot(g_ref[...][:, :D], w1b_ref[...],
                  preferred_element_type=jnp.float32)
        + par[0:1, :], 0.0)
    u = jnp.dot(h, w2_ref[...], preferred_element_type=jnp.float32)
    out_ref[...] = e + jnp.maximum(u + par[1:2, :], 0.0)


def _node_upd(emb, aggr, w1a, w1b, w2, par):
    return pl.pallas_call(
        _node_upd_body,
        grid=(N // RB,),
        in_specs=[
            pl.BlockSpec((RB, D), lambda i: (i, 0)),
            pl.BlockSpec((RB, 2 * D), lambda i: (i, 0)),
            pl.BlockSpec((D, D), lambda i: (0, 0)),
            pl.BlockSpec((D, D), lambda i: (0, 0)),
            pl.BlockSpec((D, D), lambda i: (0, 0)),
            pl.BlockSpec((8, D), lambda i: (0, 0)),
        ],
        out_specs=pl.BlockSpec((RB, D), lambda i: (i, 0)),
        out_shape=jax.ShapeDtypeStruct((N, D), jnp.float32),
    )(emb, aggr, w1a, w1b, w2, par)


def _pool_body(e_ref, par_ref, out_ref, acc_ref):
    i = pl.program_id(0)
    ssum = jnp.sum(e_ref[...], axis=0, keepdims=True)

    @pl.when(i == 0)
    def _():
        acc_ref[0:1, :] = ssum

    @pl.when(i > 0)
    def _():
        acc_ref[0:1, :] = acc_ref[0:1, :] + ssum

    @pl.when(i == N // RB - 1)
    def _():
        total = jnp.sum(acc_ref[0:1, :] * par_ref[0:1, :], keepdims=True)
        out_ref[...] = total[:, 0:1] / float(N) + par_ref[1, 0]


def _pool(emb, par):
    return pl.pallas_call(
        _pool_body,
        grid=(N // RB,),
        in_specs=[
            pl.BlockSpec((RB, D), lambda i: (i, 0)),
            pl.BlockSpec((8, D), lambda i: (0, 0)),
        ],
        out_specs=pl.BlockSpec((1, 1), lambda i: (0, 0)),
        out_shape=jax.ShapeDtypeStruct((1, 1), jnp.float32),
        scratch_shapes=[pltpu.VMEM((8, D), jnp.float32)],
    )(emb, par)


def kernel(x, edge_index, dists, batch, Win, bin_, Wm1, bm1, Wm2, bm2,
           Wu1, bu1, Wu2, bu2, Wp, bp):
    src = edge_index[0]
    dst = edge_index[1]

    xp = jnp.pad(x, ((0, 0), (0, 3)))
    wp_in = jnp.pad(Win, ((0, 3), (0, 0)))
    par_in = jnp.zeros((8, D), jnp.float32).at[0].set(bin_)

    emb = _node_in(xp, wp_in, par_in)

    for l in range(4):
        wcat = Wm1[l][:2 * D]
        wcat = jnp.concatenate([wcat[:D], wcat[D:2 * D]], axis=1)
        ab = _node_pre(emb, wcat)
        tp = _gather_add(ab, dst, src)
        par_e = (jnp.zeros((8, D), jnp.float32)
                 .at[0].set(Wm1[l][2 * D])
                 .at[1].set(bm1[l])
                 .at[2].set(bm2[l]))
        m = _edge_mlp(tp, dists, Wm2[l], par_e)
        aggr = jnp.concatenate([_scatter_q0(m, dst), _scatter_q1(m, dst)],
                               axis=0)
        par_u = (jnp.zeros((8, D), jnp.float32)
                 .at[0].set(bu1[l])
                 .at[1].set(bu2[l]))
        emb = _node_upd(emb, aggr, Wu1[l][:D], Wu1[l][D:], Wu2[l], par_u)

    par_p = (jnp.zeros((8, D), jnp.float32)
             .at[0].set(Wp[:, 0])
             .at[1, 0].set(bp[0]))
    out = _pool(emb, par_p)
    return out.reshape(-1)


# speedup vs baseline: 1.9478x; 1.2818x over previous
import functools

import jax
import jax.numpy as jnp
from jax import lax
from jax.experimental import pallas as pl
from jax.experimental.pallas import tpu as pltpu
from jax.experimental.pallas import tpu_sc as plsc

N = 50000
E = 800000
D = 64
NC, NS = 2, 16
NW = NC * NS
CH = 128
RB = 2000
EB = 2000

_SC_MESH = plsc.VectorSubcoreMesh(
    core_axis_name="c", subcore_axis_name="s", num_cores=NC, num_subcores=NS)

HALF = N // 2
RSC = 25088
TILE_ROWS = RSC // NS
GARB = 25080
ZCH = 64


def _gather_body(ab_hbm, dst_hbm, src_hbm, out_hbm,
                 idxd, idxs, bufd, bufs, bufo, gsem, osem):
    c = lax.axis_index("c")
    s = lax.axis_index("s")
    w = s * NC + c
    n_chunks = E // CH
    jmax = -(-n_chunks // NW)

    def fire(j, b):
        ch = w + j * NW

        @pl.when(ch < n_chunks)
        def _():
            base = ch * CH
            pltpu.sync_copy(dst_hbm.at[pl.ds(base, CH)], idxd.at[b])
            pltpu.sync_copy(src_hbm.at[pl.ds(base, CH)], idxs.at[b])
            pltpu.async_copy(ab_hbm.at[idxd.at[b]], bufd.at[b], gsem)
            pltpu.async_copy(ab_hbm.at[idxs.at[b]], bufs.at[b], gsem)

    def wait_gather(j, b):
        ch = w + j * NW

        @pl.when(ch < n_chunks)
        def _():
            pltpu.make_async_copy(ab_hbm.at[idxd.at[b]], bufd.at[b],
                                  gsem).wait()
            pltpu.make_async_copy(ab_hbm.at[idxs.at[b]], bufs.at[b],
                                  gsem).wait()

    def drain_out(j, b):
        ch = w + j * NW

        @pl.when((j >= 0) & (ch < n_chunks))
        def _():
            pltpu.make_async_copy(out_hbm.at[pl.ds(0, CH)], bufo.at[b],
                                  osem).wait()

    def compute_store(j, b):
        ch = w + j * NW

        @pl.when(ch < n_chunks)
        def _():
            base = ch * CH

            def row(r, cc):
                for g in range(4):
                    sl = pl.ds(g * 16, 16)
                    sh = pl.ds(D + g * 16, 16)
                    bufo[b, r, sl] = bufd[b, r, sl] + bufs[b, r, sh]
                return cc

            lax.fori_loop(0, CH, row, 0)
            pltpu.async_copy(bufo.at[b], out_hbm.at[pl.ds(base, CH)], osem)

    fire(0, 0)

    def step(j2, cc):
        for b in range(2):
            j = j2 * 2 + b
            fire(j + 1, (b + 1) % 2)
            wait_gather(j, b)
            drain_out(j - 2, b)
            compute_store(j, b)
        return cc

    lax.fori_loop(0, (jmax + 1) // 2, step, 0)
    drain_out(jmax - 2, jmax % 2)
    drain_out(jmax - 1, (jmax + 1) % 2)


_gather_add = pl.kernel(
    _gather_body,
    out_type=jax.ShapeDtypeStruct((E, D), jnp.float32),
    mesh=_SC_MESH,
    scratch_types=[
        pltpu.VMEM((2, CH), jnp.int32),
        pltpu.VMEM((2, CH), jnp.int32),
        pltpu.VMEM((2, CH, 2 * D), jnp.float32),
        pltpu.VMEM((2, CH, 2 * D), jnp.float32),
        pltpu.VMEM((2, CH, D), jnp.float32),
        pltpu.SemaphoreType.DMA,
        pltpu.SemaphoreType.DMA,
    ],
)


def _make_scatter(p):
    RQ = 12800
    GARB = 12704
    WB = 776
    CHS = 64

    def body(m_hbm, dst_hbm, out_hbm, idx, li, bufm, acc, msem):
        c = lax.axis_index("c")
        s = lax.axis_index("s")
        base_node = p * HALF + c * 12504
        own = 12504 - 8 * c
        n_chunks = E // CHS
        jmax = -(-n_chunks // NS)

        def zrow(r, cc):
            for g in range(8):
                bufm[0, r, pl.ds(g * 16, 16)] = jnp.zeros((16,), jnp.float32)
            return cc

        lax.fori_loop(0, CHS, zrow, 0)

        def zs(k, cc):
            pltpu.sync_copy(bufm.at[0],
                            acc.at[pl.ds(s * (RQ // NS) + k * CHS, CHS)])
            return cc

        lax.fori_loop(0, RQ // NS // CHS, zs, 0)
        ztail = RQ // NS - (RQ // NS // CHS) * CHS
        pltpu.sync_copy(bufm.at[0, pl.ds(0, ztail)],
                        acc.at[pl.ds(s * (RQ // NS) + (RQ // NS // CHS) * CHS,
                                     ztail)])
        plsc.subcore_barrier()

        def fire(j, b):
            ch = s + j * NS

            @pl.when(ch < n_chunks)
            def _():
                base = ch * CHS
                pltpu.sync_copy(dst_hbm.at[pl.ds(base, CHS)], idx.at[b])
                pltpu.async_copy(m_hbm.at[pl.ds(base, CHS)], bufm.at[b], msem)

        def proc(j, b):
            ch = s + j * NS

            @pl.when(ch < n_chunks)
            def _():
                pltpu.make_async_copy(m_hbm.at[pl.ds(0, CHS)], bufm.at[b],
                                      msem).wait()
                for g in range(CHS // 16):
                    sl = pl.ds(g * 16, 16)
                    v = idx[b, sl] - base_node
                    oob = (v < 0) | (v >= own)
                    li[b, sl] = jnp.where(oob, GARB, v)
                pltpu.sync_copy(bufm.at[b], acc.at[li.at[b]], add=True)

        fire(0, 0)

        def step(j2, cc):
            for b in range(2):
                j = j2 * 2 + b
                fire(j + 1, (b + 1) % 2)
                proc(j, b)
            return cc

        lax.fori_loop(0, (jmax + 1) // 2, step, 0)
        plsc.subcore_barrier()

        pltpu.sync_copy(acc.at[pl.ds(s * WB, WB)],
                        out_hbm.at[pl.ds(c * 12504 + s * WB, WB)])

        @pl.when((s == 0) & (c == 0))
        def _():
            pltpu.sync_copy(acc.at[pl.ds(NS * WB, 88)],
                            out_hbm.at[pl.ds(NS * WB, 88)])

        @pl.when((s == 0) & (c == 1))
        def _():
            pltpu.sync_copy(acc.at[pl.ds(NS * WB, 80)],
                            out_hbm.at[pl.ds(12504 + NS * WB, 80)])

    return pl.kernel(
        body,
        out_type=jax.ShapeDtypeStruct((HALF, 2 * D), jnp.float32),
        mesh=_SC_MESH,
        scratch_types=[
            pltpu.VMEM((2, 64), jnp.int32),
            pltpu.VMEM((2, 64), jnp.int32),
            pltpu.VMEM((2, 64, 2 * D), jnp.float32),
            pltpu.VMEM_SHARED((RQ, 2 * D), jnp.float32),
            pltpu.SemaphoreType.DMA,
        ],
    )


_scatter_q0 = _make_scatter(0)
_scatter_q1 = _make_scatter(1)


def _node_in_body(x_ref, w_ref, par_ref, out_ref):
    out_ref[...] = jnp.dot(x_ref[...], w_ref[...],
                           preferred_element_type=jnp.float32) + par_ref[0:1, :]


def _node_in(xp, wp, par):
    return pl.pallas_call(
        _node_in_body,
        grid=(N // RB,),
        in_specs=[
            pl.BlockSpec((RB, 8), lambda i: (i, 0)),
            pl.BlockSpec((8, D), lambda i: (0, 0)),
            pl.BlockSpec((8, D), lambda i: (0, 0)),
        ],
        out_specs=pl.BlockSpec((RB, D), lambda i: (i, 0)),
        out_shape=jax.ShapeDtypeStruct((N, D), jnp.float32),
    )(xp, wp, par)


def _node_pre_body(e_ref, w_ref, ab_ref):
    ab_ref[...] = jnp.dot(e_ref[...], w_ref[...],
                          preferred_element_type=jnp.float32)


def _node_pre(emb, wcat):
    return pl.pallas_call(
        _node_pre_body,
        grid=(N // RB,),
        in_specs=[
            pl.BlockSpec((RB, D), lambda i: (i, 0)),
            pl.BlockSpec((D, 2 * D), lambda i: (0, 0)),
        ],
        out_specs=pl.BlockSpec((RB, 2 * D), lambda i: (i, 0)),
        out_shape=jax.ShapeDtypeStruct((N, 2 * D), jnp.float32),
    )(emb, wcat)


def _edge_mlp_body(tp_ref, d_ref, w2_ref, par_ref, out_ref):
    par = par_ref[...]
    t = jnp.maximum(tp_ref[...] + d_ref[...] * par[0:1, :] + par[1:2, :], 0.0)
    m = jnp.dot(t, w2_ref[...], preferred_element_type=jnp.float32)
    m = jnp.maximum(m + par[2:3, :], 0.0)
    out_ref[...] = jnp.concatenate([m, jnp.zeros_like(m)], axis=1)


def _edge_mlp(tp, dists, wm2, par):
    return pl.pallas_call(
        _edge_mlp_body,
        grid=(E // EB,),
        in_specs=[
            pl.BlockSpec((EB, D), lambda i: (i, 0)),
            pl.BlockSpec((EB, 1), lambda i: (i, 0)),
            pl.BlockSpec((D, D), lambda i: (0, 0)),
            pl.BlockSpec((8, D), lambda i: (0, 0)),
        ],
        out_specs=pl.BlockSpec((EB, 2 * D), lambda i: (i, 0)),
        out_shape=jax.ShapeDtypeStruct((E, 2 * D), jnp.float32),
    )(tp, dists, wm2, par)


def _node_upd_body(e_ref, g_ref, w1a_ref, w1b_ref, w2_ref, par_ref, out_ref):
    par = par_ref[...]
    e = e_ref[...]
    h = jnp.maximum(
        jnp.dot(e, w1a_ref[...], preferred_element_type=jnp.float32)
        + jnp.dot(g_ref[...][:, :D], w1b_ref[...],
                  preferred_element_type=jnp.float32)
        + par[0:1, :], 0.0)
    u = jnp.dot(h, w2_ref[...], preferred_element_type=jnp.float32)
    out_ref[...] = e + jnp.maximum(u + par[1:2, :], 0.0)


def _node_upd(emb, aggr, w1a, w1b, w2, par):
    return pl.pallas_call(
        _node_upd_body,
        grid=(N // RB,),
        in_specs=[
            pl.BlockSpec((RB, D), lambda i: (i, 0)),
            pl.BlockSpec((RB, 2 * D), lambda i: (i, 0)),
            pl.BlockSpec((D, D), lambda i: (0, 0)),
            pl.BlockSpec((D, D), lambda i: (0, 0)),
            pl.BlockSpec((D, D), lambda i: (0, 0)),
            pl.BlockSpec((8, D), lambda i: (0, 0)),
        ],
        out_specs=pl.BlockSpec((RB, D), lambda i: (i, 0)),
        out_shape=jax.ShapeDtypeStruct((N, D), jnp.float32),
    )(emb, aggr, w1a, w1b, w2, par)


def _pool_body(e_ref, par_ref, out_ref, acc_ref):
    i = pl.program_id(0)
    ssum = jnp.sum(e_ref[...], axis=0, keepdims=True)

    @pl.when(i == 0)
    def _():
        acc_ref[0:1, :] = ssum

    @pl.when(i > 0)
    def _():
        acc_ref[0:1, :] = acc_ref[0:1, :] + ssum

    @pl.when(i == N // RB - 1)
    def _():
        total = jnp.sum(acc_ref[0:1, :] * par_ref[0:1, :], keepdims=True)
        out_ref[...] = total[:, 0:1] / float(N) + par_ref[1, 0]


def _pool(emb, par):
    return pl.pallas_call(
        _pool_body,
        grid=(N // RB,),
        in_specs=[
            pl.BlockSpec((RB, D), lambda i: (i, 0)),
            pl.BlockSpec((8, D), lambda i: (0, 0)),
        ],
        out_specs=pl.BlockSpec((1, 1), lambda i: (0, 0)),
        out_shape=jax.ShapeDtypeStruct((1, 1), jnp.float32),
        scratch_shapes=[pltpu.VMEM((8, D), jnp.float32)],
    )(emb, par)


def kernel(x, edge_index, dists, batch, Win, bin_, Wm1, bm1, Wm2, bm2,
           Wu1, bu1, Wu2, bu2, Wp, bp):
    src = edge_index[0]
    dst = edge_index[1]

    xp = jnp.pad(x, ((0, 0), (0, 3)))
    wp_in = jnp.pad(Win, ((0, 3), (0, 0)))
    par_in = jnp.zeros((8, D), jnp.float32).at[0].set(bin_)

    emb = _node_in(xp, wp_in, par_in)

    for l in range(4):
        wcat = Wm1[l][:2 * D]
        wcat = jnp.concatenate([wcat[:D], wcat[D:2 * D]], axis=1)
        ab = _node_pre(emb, wcat)
        tp = _gather_add(ab, dst, src)
        par_e = (jnp.zeros((8, D), jnp.float32)
                 .at[0].set(Wm1[l][2 * D])
                 .at[1].set(bm1[l])
                 .at[2].set(bm2[l]))
        m = _edge_mlp(tp, dists, Wm2[l], par_e)
        aggr = jnp.concatenate([_scatter_q0(m, dst), _scatter_q1(m, dst)],
                               axis=0)
        par_u = (jnp.zeros((8, D), jnp.float32)
                 .at[0].set(bu1[l])
                 .at[1].set(bu2[l]))
        emb = _node_upd(emb, aggr, Wu1[l][:D], Wu1[l][D:], Wu2[l], par_u)

    par_p = (jnp.zeros((8, D), jnp.float32)
             .at[0].set(Wp[:, 0])
             .at[1, 0].set(bp[0]))
    out = _pool(emb, par_p)
    return out.reshape(-1)


# speedup vs baseline: 1.9954x; 1.0244x over previous
import functools

import jax
import jax.numpy as jnp
from jax import lax
from jax.experimental import pallas as pl
from jax.experimental.pallas import tpu as pltpu
from jax.experimental.pallas import tpu_sc as plsc

N = 50000
E = 800000
D = 64
NC, NS = 2, 16
NW = NC * NS
CH = 128
RB = 2000
EB = 2000

_SC_MESH = plsc.VectorSubcoreMesh(
    core_axis_name="c", subcore_axis_name="s", num_cores=NC, num_subcores=NS)

HALF = N // 2
RSC = 25088
TILE_ROWS = RSC // NS
GARB = 25080
ZCH = 64


def _gather_body(ab_hbm, dst_hbm, src_hbm, out_hbm,
                 idxd, idxs, bufd, bufs, bufo, gsem, osem, isa, isb):
    c = lax.axis_index("c")
    s = lax.axis_index("s")
    w = s * NC + c
    n_chunks = E // CH
    jmax = -(-n_chunks // NW)
    isems = (isa, isb)

    def fire_idx(j, b4, sem):
        ch = w + j * NW

        @pl.when(ch < n_chunks)
        def _():
            base = ch * CH
            pltpu.async_copy(dst_hbm.at[pl.ds(base, CH)], idxd.at[b4], sem)
            pltpu.async_copy(src_hbm.at[pl.ds(base, CH)], idxs.at[b4], sem)

    def fire_gather(j, b4, b2, sem):
        ch = w + j * NW

        @pl.when(ch < n_chunks)
        def _():
            pltpu.make_async_copy(dst_hbm.at[pl.ds(0, CH)], idxd.at[b4],
                                  sem).wait()
            pltpu.make_async_copy(src_hbm.at[pl.ds(0, CH)], idxs.at[b4],
                                  sem).wait()
            pltpu.async_copy(ab_hbm.at[idxd.at[b4]], bufd.at[b2], gsem)
            pltpu.async_copy(ab_hbm.at[idxs.at[b4]], bufs.at[b2], gsem)

    def wait_gather(j, b2):
        ch = w + j * NW

        @pl.when(ch < n_chunks)
        def _():
            pltpu.make_async_copy(ab_hbm.at[idxd.at[0]], bufd.at[b2],
                                  gsem).wait()
            pltpu.make_async_copy(ab_hbm.at[idxs.at[0]], bufs.at[b2],
                                  gsem).wait()

    def drain_out(j, b2):
        ch = w + j * NW

        @pl.when((j >= 0) & (ch < n_chunks))
        def _():
            pltpu.make_async_copy(out_hbm.at[pl.ds(0, CH)], bufo.at[b2],
                                  osem).wait()

    def compute_store(j, b2):
        ch = w + j * NW

        @pl.when(ch < n_chunks)
        def _():
            base = ch * CH

            def row(r, cc):
                for g in range(4):
                    sl = pl.ds(g * 16, 16)
                    sh = pl.ds(D + g * 16, 16)
                    bufo[b2, r, sl] = bufd[b2, r, sl] + bufs[b2, r, sh]
                return cc

            lax.fori_loop(0, CH, row, 0)
            pltpu.async_copy(bufo.at[b2], out_hbm.at[pl.ds(base, CH)], osem)

    fire_idx(0, 0, isems[0])
    fire_idx(1, 1, isems[1])
    fire_gather(0, 0, 0, isems[0])

    def step(j4, cc):
        for b in range(4):
            j = j4 * 4 + b
            fire_idx(j + 2, (b + 2) % 4, isems[b % 2])
            fire_gather(j + 1, (b + 1) % 4, (b + 1) % 2, isems[(b + 1) % 2])
            wait_gather(j, b % 2)
            drain_out(j - 2, b % 2)
            compute_store(j, b % 2)
        return cc

    lax.fori_loop(0, (jmax + 3) // 4, step, 0)
    drain_out(jmax - 2, jmax % 2)
    drain_out(jmax - 1, (jmax + 1) % 2)


_gather_add = pl.kernel(
    _gather_body,
    out_type=jax.ShapeDtypeStruct((E, D), jnp.float32),
    mesh=_SC_MESH,
    scratch_types=[
        pltpu.VMEM((4, CH), jnp.int32),
        pltpu.VMEM((4, CH), jnp.int32),
        pltpu.VMEM((2, CH, 2 * D), jnp.float32),
        pltpu.VMEM((2, CH, 2 * D), jnp.float32),
        pltpu.VMEM((2, CH, D), jnp.float32),
        pltpu.SemaphoreType.DMA,
        pltpu.SemaphoreType.DMA,
        pltpu.SemaphoreType.DMA,
        pltpu.SemaphoreType.DMA,
    ],
)


def _make_scatter(p):
    RQ = 12800
    GARB = 12704
    WB = 776
    CHS = 64

    def body(m_hbm, dst_hbm, out_hbm, idx, li, bufm, acc, msem, isa, isb):
        c = lax.axis_index("c")
        s = lax.axis_index("s")
        base_node = p * HALF + c * 12504
        own = 12504 - 8 * c
        n_chunks = E // CHS
        jmax = -(-n_chunks // NS)

        def zrow(r, cc):
            for g in range(8):
                bufm[0, r, pl.ds(g * 16, 16)] = jnp.zeros((16,), jnp.float32)
                bufm[1, r, pl.ds(g * 16, 16)] = jnp.zeros((16,), jnp.float32)
            return cc

        lax.fori_loop(0, CHS, zrow, 0)

        def zs(k, cc):
            pltpu.sync_copy(bufm.at[0],
                            acc.at[pl.ds(s * (RQ // NS) + k * CHS, CHS)])
            return cc

        lax.fori_loop(0, RQ // NS // CHS, zs, 0)
        ztail = RQ // NS - (RQ // NS // CHS) * CHS
        pltpu.sync_copy(bufm.at[0, pl.ds(0, ztail)],
                        acc.at[pl.ds(s * (RQ // NS) + (RQ // NS // CHS) * CHS,
                                     ztail)])
        plsc.subcore_barrier()

        isems = (isa, isb)

        def fire_idx(j, b4, sem):
            ch = s + j * NS

            @pl.when(ch < n_chunks)
            def _():
                pltpu.async_copy(dst_hbm.at[pl.ds(ch * CHS, CHS)],
                                 idx.at[b4], sem)

        def fire_m(j, b2):
            ch = s + j * NS

            @pl.when(ch < n_chunks)
            def _():
                pltpu.async_copy(m_hbm.at[pl.ds(ch * CHS, CHS)],
                                 bufm.at[b2], msem)

        def proc(j, b4, b2, sem):
            ch = s + j * NS

            @pl.when(ch < n_chunks)
            def _():
                pltpu.make_async_copy(dst_hbm.at[pl.ds(0, CHS)], idx.at[b4],
                                      sem).wait()
                pltpu.make_async_copy(m_hbm.at[pl.ds(0, CHS)], bufm.at[b2],
                                      msem).wait()
                for g in range(CHS // 16):
                    sl = pl.ds(g * 16, 16)
                    v = idx[b4, sl] - base_node
                    oob = (v < 0) | (v >= own)
                    li[b2, sl] = jnp.where(oob, GARB, v)
                pltpu.sync_copy(bufm.at[b2], acc.at[li.at[b2]], add=True)

        fire_idx(0, 0, isems[0])
        fire_idx(1, 1, isems[1])
        fire_m(0, 0)

        def step(j4, cc):
            for b in range(4):
                j = j4 * 4 + b
                fire_idx(j + 2, (b + 2) % 4, isems[b % 2])
                fire_m(j + 1, (b + 1) % 2)
                proc(j, b % 4, b % 2, isems[b % 2])
            return cc

        lax.fori_loop(0, (jmax + 3) // 4, step, 0)
        plsc.subcore_barrier()

        pltpu.sync_copy(acc.at[pl.ds(s * WB, WB)],
                        out_hbm.at[pl.ds(c * 12504 + s * WB, WB)])

        @pl.when((s == 0) & (c == 0))
        def _():
            pltpu.sync_copy(acc.at[pl.ds(NS * WB, 88)],
                            out_hbm.at[pl.ds(NS * WB, 88)])

        @pl.when((s == 0) & (c == 1))
        def _():
            pltpu.sync_copy(acc.at[pl.ds(NS * WB, 80)],
                            out_hbm.at[pl.ds(12504 + NS * WB, 80)])

    return pl.kernel(
        body,
        out_type=jax.ShapeDtypeStruct((HALF, 2 * D), jnp.float32),
        mesh=_SC_MESH,
        scratch_types=[
            pltpu.VMEM((4, 64), jnp.int32),
            pltpu.VMEM((2, 64), jnp.int32),
            pltpu.VMEM((2, 64, 2 * D), jnp.float32),
            pltpu.VMEM_SHARED((RQ, 2 * D), jnp.float32),
            pltpu.SemaphoreType.DMA,
            pltpu.SemaphoreType.DMA,
            pltpu.SemaphoreType.DMA,
        ],
    )


_scatter_q0 = _make_scatter(0)
_scatter_q1 = _make_scatter(1)


def _node_in_body(x_ref, w_ref, par_ref, out_ref):
    out_ref[...] = jnp.dot(x_ref[...], w_ref[...],
                           preferred_element_type=jnp.float32) + par_ref[0:1, :]


def _node_in(xp, wp, par):
    return pl.pallas_call(
        _node_in_body,
        grid=(N // RB,),
        in_specs=[
            pl.BlockSpec((RB, 8), lambda i: (i, 0)),
            pl.BlockSpec((8, D), lambda i: (0, 0)),
            pl.BlockSpec((8, D), lambda i: (0, 0)),
        ],
        out_specs=pl.BlockSpec((RB, D), lambda i: (i, 0)),
        out_shape=jax.ShapeDtypeStruct((N, D), jnp.float32),
    )(xp, wp, par)


def _node_pre_body(e_ref, w_ref, ab_ref):
    ab_ref[...] = jnp.dot(e_ref[...], w_ref[...],
                          preferred_element_type=jnp.float32)


def _node_pre(emb, wcat):
    return pl.pallas_call(
        _node_pre_body,
        grid=(N // RB,),
        in_specs=[
            pl.BlockSpec((RB, D), lambda i: (i, 0)),
            pl.BlockSpec((D, 2 * D), lambda i: (0, 0)),
        ],
        out_specs=pl.BlockSpec((RB, 2 * D), lambda i: (i, 0)),
        out_shape=jax.ShapeDtypeStruct((N, 2 * D), jnp.float32),
    )(emb, wcat)


def _edge_mlp_body(tp_ref, d_ref, w2_ref, par_ref, out_ref):
    par = par_ref[...]
    t = jnp.maximum(tp_ref[...] + d_ref[...] * par[0:1, :] + par[1:2, :], 0.0)
    m = jnp.dot(t, w2_ref[...], preferred_element_type=jnp.float32)
    m = jnp.maximum(m + par[2:3, :], 0.0)
    out_ref[...] = jnp.concatenate([m, jnp.zeros_like(m)], axis=1)


def _edge_mlp(tp, dists, wm2, par):
    return pl.pallas_call(
        _edge_mlp_body,
        grid=(E // EB,),
        in_specs=[
            pl.BlockSpec((EB, D), lambda i: (i, 0)),
            pl.BlockSpec((EB, 1), lambda i: (i, 0)),
            pl.BlockSpec((D, D), lambda i: (0, 0)),
            pl.BlockSpec((8, D), lambda i: (0, 0)),
        ],
        out_specs=pl.BlockSpec((EB, 2 * D), lambda i: (i, 0)),
        out_shape=jax.ShapeDtypeStruct((E, 2 * D), jnp.float32),
    )(tp, dists, wm2, par)


def _node_upd_body(e_ref, g_ref, w1a_ref, w1b_ref, w2_ref, par_ref, out_ref):
    par = par_ref[...]
    e = e_ref[...]
    h = jnp.maximum(
        jnp.dot(e, w1a_ref[...], preferred_element_type=jnp.float32)
        + jnp.dot(g_ref[...][:, :D], w1b_ref[...],
                  preferred_element_type=jnp.float32)
        + par[0:1, :], 0.0)
    u = jnp.dot(h, w2_ref[...], preferred_element_type=jnp.float32)
    out_ref[...] = e + jnp.maximum(u + par[1:2, :], 0.0)


def _node_upd(emb, aggr, w1a, w1b, w2, par):
    return pl.pallas_call(
        _node_upd_body,
        grid=(N // RB,),
        in_specs=[
            pl.BlockSpec((RB, D), lambda i: (i, 0)),
            pl.BlockSpec((RB, 2 * D), lambda i: (i, 0)),
            pl.BlockSpec((D, D), lambda i: (0, 0)),
            pl.BlockSpec((D, D), lambda i: (0, 0)),
            pl.BlockSpec((D, D), lambda i: (0, 0)),
            pl.BlockSpec((8, D), lambda i: (0, 0)),
        ],
        out_specs=pl.BlockSpec((RB, D), lambda i: (i, 0)),
        out_shape=jax.ShapeDtypeStruct((N, D), jnp.float32),
    )(emb, aggr, w1a, w1b, w2, par)


def _pool_body(e_ref, par_ref, out_ref, acc_ref):
    i = pl.program_id(0)
    ssum = jnp.sum(e_ref[...], axis=0, keepdims=True)

    @pl.when(i == 0)
    def _():
        acc_ref[0:1, :] = ssum

    @pl.when(i > 0)
    def _():
        acc_ref[0:1, :] = acc_ref[0:1, :] + ssum

    @pl.when(i == N // RB - 1)
    def _():
        total = jnp.sum(acc_ref[0:1, :] * par_ref[0:1, :], keepdims=True)
        out_ref[...] = total[:, 0:1] / float(N) + par_ref[1, 0]


def _pool(emb, par):
    return pl.pallas_call(
        _pool_body,
        grid=(N // RB,),
        in_specs=[
            pl.BlockSpec((RB, D), lambda i: (i, 0)),
            pl.BlockSpec((8, D), lambda i: (0, 0)),
        ],
        out_specs=pl.BlockSpec((1, 1), lambda i: (0, 0)),
        out_shape=jax.ShapeDtypeStruct((1, 1), jnp.float32),
        scratch_shapes=[pltpu.VMEM((8, D), jnp.float32)],
    )(emb, par)


def kernel(x, edge_index, dists, batch, Win, bin_, Wm1, bm1, Wm2, bm2,
           Wu1, bu1, Wu2, bu2, Wp, bp):
    src = edge_index[0]
    dst = edge_index[1]

    xp = jnp.pad(x, ((0, 0), (0, 3)))
    wp_in = jnp.pad(Win, ((0, 3), (0, 0)))
    par_in = jnp.zeros((8, D), jnp.float32).at[0].set(bin_)

    emb = _node_in(xp, wp_in, par_in)

    for l in range(4):
        wcat = Wm1[l][:2 * D]
        wcat = jnp.concatenate([wcat[:D], wcat[D:2 * D]], axis=1)
        ab = _node_pre(emb, wcat)
        tp = _gather_add(ab, dst, src)
        par_e = (jnp.zeros((8, D), jnp.float32)
                 .at[0].set(Wm1[l][2 * D])
                 .at[1].set(bm1[l])
                 .at[2].set(bm2[l]))
        m = _edge_mlp(tp, dists, Wm2[l], par_e)
        aggr = jnp.concatenate([_scatter_q0(m, dst), _scatter_q1(m, dst)],
                               axis=0)
        par_u = (jnp.zeros((8, D), jnp.float32)
                 .at[0].set(bu1[l])
                 .at[1].set(bu2[l]))
        emb = _node_upd(emb, aggr, Wu1[l][:D], Wu1[l][D:], Wu2[l], par_u)

    par_p = (jnp.zeros((8, D), jnp.float32)
             .at[0].set(Wp[:, 0])
             .at[1, 0].set(bp[0]))
    out = _pool(emb, par_p)
    return out.reshape(-1)


# speedup vs baseline: 1.9977x; 1.0011x over previous
import jax
import jax.numpy as jnp
from jax import lax
from jax.experimental import pallas as pl
from jax.experimental.pallas import tpu as pltpu
from jax.experimental.pallas import tpu_sc as plsc

N = 50000
E = 800000
D = 64
NC, NS = 2, 16
NW = NC * NS
CH = 128
RB = 2000
EB = 2000

_SC_MESH = plsc.VectorSubcoreMesh(
    core_axis_name="c", subcore_axis_name="s", num_cores=NC, num_subcores=NS)

HALF = N // 2


def _gather_body(ab_hbm, dst_hbm, src_hbm, out_hbm,
                 idxd, idxs, bufd, bufs, bufo, gsem, osem, isa, isb):
    c = lax.axis_index("c")
    s = lax.axis_index("s")
    w = s * NC + c
    n_chunks = E // CH
    jmax = -(-n_chunks // NW)
    isems = (isa, isb)

    def fire_idx(j, b4, sem):
        ch = w + j * NW

        @pl.when(ch < n_chunks)
        def _():
            base = ch * CH
            pltpu.async_copy(dst_hbm.at[pl.ds(base, CH)], idxd.at[b4], sem)
            pltpu.async_copy(src_hbm.at[pl.ds(base, CH)], idxs.at[b4], sem)

    def fire_gather(j, b4, b2, sem):
        ch = w + j * NW

        @pl.when(ch < n_chunks)
        def _():
            pltpu.make_async_copy(dst_hbm.at[pl.ds(0, CH)], idxd.at[b4],
                                  sem).wait()
            pltpu.make_async_copy(src_hbm.at[pl.ds(0, CH)], idxs.at[b4],
                                  sem).wait()
            pltpu.async_copy(ab_hbm.at[idxd.at[b4]], bufd.at[b2], gsem)
            pltpu.async_copy(ab_hbm.at[idxs.at[b4]], bufs.at[b2], gsem)

    def wait_gather(j, b2):
        ch = w + j * NW

        @pl.when(ch < n_chunks)
        def _():
            pltpu.make_async_copy(ab_hbm.at[idxd.at[0]], bufd.at[b2],
                                  gsem).wait()
            pltpu.make_async_copy(ab_hbm.at[idxs.at[0]], bufs.at[b2],
                                  gsem).wait()

    def drain_out(j, b2):
        ch = w + j * NW

        @pl.when((j >= 0) & (ch < n_chunks))
        def _():
            pltpu.make_async_copy(out_hbm.at[pl.ds(0, CH)], bufo.at[b2],
                                  osem).wait()

    def compute_store(j, b2):
        ch = w + j * NW

        @pl.when(ch < n_chunks)
        def _():
            base = ch * CH

            def row(r, cc):
                for g in range(4):
                    sl = pl.ds(g * 16, 16)
                    sh = pl.ds(D + g * 16, 16)
                    bufo[b2, r, sl] = bufd[b2, r, sl] + bufs[b2, r, sh]
                return cc

            lax.fori_loop(0, CH, row, 0)
            pltpu.async_copy(bufo.at[b2], out_hbm.at[pl.ds(base, CH)], osem)

    fire_idx(0, 0, isems[0])
    fire_idx(1, 1, isems[1])
    fire_gather(0, 0, 0, isems[0])

    def step(j4, cc):
        for b in range(4):
            j = j4 * 4 + b
            fire_idx(j + 2, (b + 2) % 4, isems[b % 2])
            fire_gather(j + 1, (b + 1) % 4, (b + 1) % 2, isems[(b + 1) % 2])
            wait_gather(j, b % 2)
            drain_out(j - 2, b % 2)
            compute_store(j, b % 2)
        return cc

    lax.fori_loop(0, (jmax + 3) // 4, step, 0)
    drain_out(jmax - 2, jmax % 2)
    drain_out(jmax - 1, (jmax + 1) % 2)


_gather_add = pl.kernel(
    _gather_body,
    out_type=jax.ShapeDtypeStruct((E, D), jnp.float32),
    mesh=_SC_MESH,
    scratch_types=[
        pltpu.VMEM((4, CH), jnp.int32),
        pltpu.VMEM((4, CH), jnp.int32),
        pltpu.VMEM((2, CH, 2 * D), jnp.float32),
        pltpu.VMEM((2, CH, 2 * D), jnp.float32),
        pltpu.VMEM((2, CH, D), jnp.float32),
        pltpu.SemaphoreType.DMA,
        pltpu.SemaphoreType.DMA,
        pltpu.SemaphoreType.DMA,
        pltpu.SemaphoreType.DMA,
    ],
)


def _make_scatter(p):
    RQ = 12800
    GARB = 12704
    WB = 776
    CHS = 64

    def body(m_hbm, dst_hbm, out_hbm, idx, li, bufm, acc, msem, isa, isb):
        c = lax.axis_index("c")
        s = lax.axis_index("s")
        base_node = p * HALF + c * 12504
        own = 12504 - 8 * c
        n_chunks = E // CHS
        jmax = -(-n_chunks // NS)

        def zrow(r, cc):
            for g in range(8):
                bufm[0, r, pl.ds(g * 16, 16)] = jnp.zeros((16,), jnp.float32)
                bufm[1, r, pl.ds(g * 16, 16)] = jnp.zeros((16,), jnp.float32)
            return cc

        lax.fori_loop(0, CHS, zrow, 0)

        def zs(k, cc):
            pltpu.sync_copy(bufm.at[0],
                            acc.at[pl.ds(s * (RQ // NS) + k * CHS, CHS)])
            return cc

        lax.fori_loop(0, RQ // NS // CHS, zs, 0)
        ztail = RQ // NS - (RQ // NS // CHS) * CHS
        pltpu.sync_copy(bufm.at[0, pl.ds(0, ztail)],
                        acc.at[pl.ds(s * (RQ // NS) + (RQ // NS // CHS) * CHS,
                                     ztail)])
        plsc.subcore_barrier()

        isems = (isa, isb)

        def fire_idx(j, b4, sem):
            ch = s + j * NS

            @pl.when(ch < n_chunks)
            def _():
                pltpu.async_copy(dst_hbm.at[pl.ds(ch * CHS, CHS)],
                                 idx.at[b4], sem)

        def fire_m(j, b2):
            ch = s + j * NS

            @pl.when(ch < n_chunks)
            def _():
                pltpu.async_copy(m_hbm.at[pl.ds(ch * CHS, CHS)],
                                 bufm.at[b2], msem)

        def proc(j, b4, b2, sem):
            ch = s + j * NS

            @pl.when(ch < n_chunks)
            def _():
                pltpu.make_async_copy(dst_hbm.at[pl.ds(0, CHS)], idx.at[b4],
                                      sem).wait()
                pltpu.make_async_copy(m_hbm.at[pl.ds(0, CHS)], bufm.at[b2],
                                      msem).wait()
                for g in range(CHS // 16):
                    sl = pl.ds(g * 16, 16)
                    v = idx[b4, sl] - base_node
                    oob = (v < 0) | (v >= own)
                    li[b2, sl] = jnp.where(oob, GARB, v)
                pltpu.sync_copy(bufm.at[b2], acc.at[li.at[b2]], add=True)

        fire_idx(0, 0, isems[0])
        fire_idx(1, 1, isems[1])
        fire_m(0, 0)

        def step(j4, cc):
            for b in range(4):
                j = j4 * 4 + b
                fire_idx(j + 2, (b + 2) % 4, isems[b % 2])
                fire_m(j + 1, (b + 1) % 2)
                proc(j, b % 4, b % 2, isems[b % 2])
            return cc

        lax.fori_loop(0, (jmax + 3) // 4, step, 0)
        plsc.subcore_barrier()

        pltpu.sync_copy(acc.at[pl.ds(s * WB, WB)],
                        out_hbm.at[pl.ds(c * 12504 + s * WB, WB)])

        @pl.when((s == 0) & (c == 0))
        def _():
            pltpu.sync_copy(acc.at[pl.ds(NS * WB, 88)],
                            out_hbm.at[pl.ds(NS * WB, 88)])

        @pl.when((s == 0) & (c == 1))
        def _():
            pltpu.sync_copy(acc.at[pl.ds(NS * WB, 80)],
                            out_hbm.at[pl.ds(12504 + NS * WB, 80)])

    return pl.kernel(
        body,
        out_type=jax.ShapeDtypeStruct((HALF, 2 * D), jnp.float32),
        mesh=_SC_MESH,
        scratch_types=[
            pltpu.VMEM((4, 64), jnp.int32),
            pltpu.VMEM((2, 64), jnp.int32),
            pltpu.VMEM((2, 64, 2 * D), jnp.float32),
            pltpu.VMEM_SHARED((RQ, 2 * D), jnp.float32),
            pltpu.SemaphoreType.DMA,
            pltpu.SemaphoreType.DMA,
            pltpu.SemaphoreType.DMA,
        ],
    )


_scatter_q0 = _make_scatter(0)
_scatter_q1 = _make_scatter(1)


def _node_in_body(x_ref, w_ref, par_ref, out_ref):
    out_ref[...] = jnp.dot(x_ref[...], w_ref[...],
                           preferred_element_type=jnp.float32) + par_ref[0:1, :]


def _node_in(xp, wp, par):
    return pl.pallas_call(
        _node_in_body,
        grid=(N // RB,),
        in_specs=[
            pl.BlockSpec((RB, 8), lambda i: (i, 0)),
            pl.BlockSpec((8, D), lambda i: (0, 0)),
            pl.BlockSpec((8, D), lambda i: (0, 0)),
        ],
        out_specs=pl.BlockSpec((RB, D), lambda i: (i, 0)),
        out_shape=jax.ShapeDtypeStruct((N, D), jnp.float32),
    )(xp, wp, par)


def _node_pre_body(e_ref, w_ref, ab_ref):
    ab_ref[...] = jnp.dot(e_ref[...], w_ref[...],
                          preferred_element_type=jnp.float32)


def _node_pre(emb, wcat):
    return pl.pallas_call(
        _node_pre_body,
        grid=(N // RB,),
        in_specs=[
            pl.BlockSpec((RB, D), lambda i: (i, 0)),
            pl.BlockSpec((D, 2 * D), lambda i: (0, 0)),
        ],
        out_specs=pl.BlockSpec((RB, 2 * D), lambda i: (i, 0)),
        out_shape=jax.ShapeDtypeStruct((N, 2 * D), jnp.float32),
    )(emb, wcat)


def _edge_mlp_body(tp_ref, d_ref, w2_ref, par_ref, out_ref):
    par = par_ref[...]
    t = jnp.maximum(tp_ref[...] + d_ref[...] * par[0:1, :] + par[1:2, :], 0.0)
    m = jnp.dot(t, w2_ref[...], preferred_element_type=jnp.float32)
    m = jnp.maximum(m + par[2:3, :], 0.0)
    out_ref[...] = jnp.concatenate([m, jnp.zeros_like(m)], axis=1)


def _edge_mlp(tp, dists, wm2, par):
    return pl.pallas_call(
        _edge_mlp_body,
        grid=(E // EB,),
        in_specs=[
            pl.BlockSpec((EB, D), lambda i: (i, 0)),
            pl.BlockSpec((EB, 1), lambda i: (i, 0)),
            pl.BlockSpec((D, D), lambda i: (0, 0)),
            pl.BlockSpec((8, D), lambda i: (0, 0)),
        ],
        out_specs=pl.BlockSpec((EB, 2 * D), lambda i: (i, 0)),
        out_shape=jax.ShapeDtypeStruct((E, 2 * D), jnp.float32),
    )(tp, dists, wm2, par)


def _node_upd_body(e_ref, g_ref, w1a_ref, w1b_ref, w2_ref, par_ref, out_ref):
    par = par_ref[...]
    e = e_ref[...]
    h = jnp.maximum(
        jnp.dot(e, w1a_ref[...], preferred_element_type=jnp.float32)
        + jnp.dot(g_ref[...][:, :D], w1b_ref[...],
                  preferred_element_type=jnp.float32)
        + par[0:1, :], 0.0)
    u = jnp.dot(h, w2_ref[...], preferred_element_type=jnp.float32)
    out_ref[...] = e + jnp.maximum(u + par[1:2, :], 0.0)


def _node_upd(emb, aggr, w1a, w1b, w2, par):
    return pl.pallas_call(
        _node_upd_body,
        grid=(N // RB,),
        in_specs=[
            pl.BlockSpec((RB, D), lambda i: (i, 0)),
            pl.BlockSpec((RB, 2 * D), lambda i: (i, 0)),
            pl.BlockSpec((D, D), lambda i: (0, 0)),
            pl.BlockSpec((D, D), lambda i: (0, 0)),
            pl.BlockSpec((D, D), lambda i: (0, 0)),
            pl.BlockSpec((8, D), lambda i: (0, 0)),
        ],
        out_specs=pl.BlockSpec((RB, D), lambda i: (i, 0)),
        out_shape=jax.ShapeDtypeStruct((N, D), jnp.float32),
    )(emb, aggr, w1a, w1b, w2, par)


def _pool_body(e_ref, par_ref, out_ref, acc_ref):
    i = pl.program_id(0)
    ssum = jnp.sum(e_ref[...], axis=0, keepdims=True)

    @pl.when(i == 0)
    def _():
        acc_ref[0:1, :] = ssum

    @pl.when(i > 0)
    def _():
        acc_ref[0:1, :] = acc_ref[0:1, :] + ssum

    @pl.when(i == N // RB - 1)
    def _():
        total = jnp.sum(acc_ref[0:1, :] * par_ref[0:1, :], keepdims=True)
        out_ref[...] = total[:, 0:1] / float(N) + par_ref[1, 0]


def _pool(emb, par):
    return pl.pallas_call(
        _pool_body,
        grid=(N // RB,),
        in_specs=[
            pl.BlockSpec((RB, D), lambda i: (i, 0)),
            pl.BlockSpec((8, D), lambda i: (0, 0)),
        ],
        out_specs=pl.BlockSpec((1, 1), lambda i: (0, 0)),
        out_shape=jax.ShapeDtypeStruct((1, 1), jnp.float32),
        scratch_shapes=[pltpu.VMEM((8, D), jnp.float32)],
    )(emb, par)


def kernel(x, edge_index, dists, batch, Win, bin_, Wm1, bm1, Wm2, bm2,
           Wu1, bu1, Wu2, bu2, Wp, bp):
    src = edge_index[0]
    dst = edge_index[1]

    xp = jnp.pad(x, ((0, 0), (0, 3)))
    wp_in = jnp.pad(Win, ((0, 3), (0, 0)))
    par_in = jnp.zeros((8, D), jnp.float32).at[0].set(bin_)

    emb = _node_in(xp, wp_in, par_in)

    for l in range(4):
        wcat = Wm1[l][:2 * D]
        wcat = jnp.concatenate([wcat[:D], wcat[D:2 * D]], axis=1)
        ab = _node_pre(emb, wcat)
        tp = _gather_add(ab, dst, src)
        par_e = (jnp.zeros((8, D), jnp.float32)
                 .at[0].set(Wm1[l][2 * D])
                 .at[1].set(bm1[l])
                 .at[2].set(bm2[l]))
        m = _edge_mlp(tp, dists, Wm2[l], par_e)
        aggr = jnp.concatenate([_scatter_q0(m, dst), _scatter_q1(m, dst)],
                               axis=0)
        par_u = (jnp.zeros((8, D), jnp.float32)
                 .at[0].set(bu1[l])
                 .at[1].set(bu2[l]))
        emb = _node_upd(emb, aggr, Wu1[l][:D], Wu1[l][D:], Wu2[l], par_u)

    par_p = (jnp.zeros((8, D), jnp.float32)
             .at[0].set(Wp[:, 0])
             .at[1, 0].set(bp[0]))
    out = _pool(emb, par_p)
    return out.reshape(-1)
